# baseline (speedup 1.0000x reference)
import jax
import jax.numpy as jnp
from jax.experimental import pallas as pl
from jax.experimental.pallas import tpu as pltpu

_N, _F, _H, _L, _T = 100000, 128, 64, 4, 5
_BNR = 5000
_STEPS = (_N // 2) // _BNR


def _fused_kernel(xA_ref, xB_ref, Wn_ref, bn_ref, attW_ref, attb_ref,
                  readW_ref, readb_ref, h1W_ref, h1b_ref, h2W_ref, h2b_ref,
                  o0_ref, o1_ref, o2_ref, o3_ref, o4_ref, acc_ref):
    step = pl.program_id(0)

    @pl.when(step == 0)
    def _init():
        acc_ref[...] = jnp.zeros_like(acc_ref)

    x2 = jnp.concatenate([xA_ref[...], xB_ref[...]], axis=1)
    h = jnp.dot(x2, Wn_ref[...],
                preferred_element_type=jnp.float32) + bn_ref[...]
    for i in range(_L):
        h = jnp.dot(h, attW_ref[i], preferred_element_type=jnp.float32)
        h = jnp.maximum(h + attb_ref[i:i + 1, :], 0.0)
        acc_ref[i:i + 1, :] += jnp.sum(h, axis=0, keepdims=True)

    @pl.when(step == _STEPS - 1)
    def _readout():
        sums = acc_ref[:, :_H] + acc_ref[:, _H:]
        pooled = sums * (1.0 / _N)
        gr = jnp.sum(readb_ref[...], axis=0, keepdims=True)
        for i in range(_L):
            gr = gr + jnp.dot(pooled[i:i + 1, :], readW_ref[i],
                              preferred_element_type=jnp.float32)
        outs = (o0_ref, o1_ref, o2_ref, o3_ref, o4_ref)
        for j in range(_T):
            z = jnp.dot(gr, h1W_ref[j], preferred_element_type=jnp.float32)
            z = jnp.maximum(z + h1b_ref[j:j + 1, :], 0.0)
            o = (jnp.sum(z * h2W_ref[j:j + 1, :], axis=1, keepdims=True)
                 + h2b_ref[0:1, j:j + 1])
            if j in (0, 3, 4):
                o = jax.nn.sigmoid(o)
            outs[j][...] = o


def kernel(x, Wn, bn, att_W, att_b, read_W, read_b, h1_W, h1_b, h2_W, h2_b):
    eye2 = jnp.eye(2, dtype=x.dtype)
    Wn2 = jnp.kron(eye2, Wn)
    bn2 = jnp.tile(bn, 2).reshape(1, 2 * _H)
    att_W2 = jax.vmap(lambda w: jnp.kron(eye2, w))(att_W)
    att_b2 = jnp.tile(att_b, (1, 2))
    h2_Ws = h2_W[:, :, 0]
    h2_bs = h2_b.reshape(1, _T)

    whole = lambda a: pl.BlockSpec(a.shape, lambda i: (0,) * a.ndim)
    out_shape = tuple(jax.ShapeDtypeStruct((1, 1), jnp.float32)
                      for _ in range(_T))
    out_specs = tuple(pl.BlockSpec((1, 1), lambda i: (0, 0))
                      for _ in range(_T))

    return pl.pallas_call(
        _fused_kernel,
        grid=(_STEPS,),
        in_specs=[
            pl.BlockSpec((_BNR, _F), lambda i: (i, 0)),
            pl.BlockSpec((_BNR, _F), lambda i: (i + _STEPS, 0)),
            whole(Wn2), whole(bn2), whole(att_W2), whole(att_b2),
            whole(read_W), whole(read_b), whole(h1_W), whole(h1_b),
            whole(h2_Ws), whole(h2_bs),
        ],
        out_specs=out_specs,
        out_shape=out_shape,
        scratch_shapes=[pltpu.VMEM((_L, 2 * _H), jnp.float32)],
        compiler_params=pltpu.CompilerParams(
            dimension_semantics=("arbitrary",)),
    )(x, x, Wn2, bn2, att_W2, att_b2, read_W, read_b, h1_W, h1_b,
      h2_Ws, h2_bs)


# speedup vs baseline: 1.1086x; 1.1086x over previous
import jax
import jax.numpy as jnp
from jax.experimental import pallas as pl
from jax.experimental.pallas import tpu as pltpu

_N, _F, _H, _L, _T = 100000, 128, 64, 4, 5
_BNR = 2000
_STEPS = (_N // 2) // _BNR


def _fused_kernel(xA_ref, xB_ref, Wn_ref, bn_ref, attW_ref, attb_ref,
                  readW_ref, readb_ref, h1W_ref, h1b_ref, h2W_ref, h2b_ref,
                  o0_ref, o1_ref, o2_ref, o3_ref, o4_ref, acc_ref):
    step = pl.program_id(0)

    @pl.when(step == 0)
    def _init():
        acc_ref[...] = jnp.zeros_like(acc_ref)

    x2 = jnp.concatenate([xA_ref[...], xB_ref[...]], axis=1)
    h = jnp.dot(x2, Wn_ref[...],
                preferred_element_type=jnp.float32) + bn_ref[...]
    for i in range(_L):
        h = jnp.dot(h, attW_ref[i], preferred_element_type=jnp.float32)
        h = jnp.maximum(h + attb_ref[i:i + 1, :], 0.0)
        acc_ref[i:i + 1, :] += jnp.sum(h, axis=0, keepdims=True)

    @pl.when(step == _STEPS - 1)
    def _readout():
        sums = acc_ref[:, :_H] + acc_ref[:, _H:]
        pooled = sums * (1.0 / _N)
        gr = jnp.sum(readb_ref[...], axis=0, keepdims=True)
        for i in range(_L):
            gr = gr + jnp.dot(pooled[i:i + 1, :], readW_ref[i],
                              preferred_element_type=jnp.float32)
        outs = (o0_ref, o1_ref, o2_ref, o3_ref, o4_ref)
        for j in range(_T):
            z = jnp.dot(gr, h1W_ref[j], preferred_element_type=jnp.float32)
            z = jnp.maximum(z + h1b_ref[j:j + 1, :], 0.0)
            o = (jnp.sum(z * h2W_ref[j:j + 1, :], axis=1, keepdims=True)
                 + h2b_ref[0:1, j:j + 1])
            if j in (0, 3, 4):
                o = jax.nn.sigmoid(o)
            outs[j][...] = o


def kernel(x, Wn, bn, att_W, att_b, read_W, read_b, h1_W, h1_b, h2_W, h2_b):
    eye2 = jnp.eye(2, dtype=x.dtype)
    Wn2 = jnp.kron(eye2, Wn)
    bn2 = jnp.tile(bn, 2).reshape(1, 2 * _H)
    att_W2 = jax.vmap(lambda w: jnp.kron(eye2, w))(att_W)
    att_b2 = jnp.tile(att_b, (1, 2))
    h2_Ws = h2_W[:, :, 0]
    h2_bs = h2_b.reshape(1, _T)

    whole = lambda a: pl.BlockSpec(a.shape, lambda i: (0,) * a.ndim)
    out_shape = tuple(jax.ShapeDtypeStruct((1, 1), jnp.float32)
                      for _ in range(_T))
    out_specs = tuple(pl.BlockSpec((1, 1), lambda i: (0, 0))
                      for _ in range(_T))

    return pl.pallas_call(
        _fused_kernel,
        grid=(_STEPS,),
        in_specs=[
            pl.BlockSpec((_BNR, _F), lambda i: (i, 0)),
            pl.BlockSpec((_BNR, _F), lambda i: (i + _STEPS, 0)),
            whole(Wn2), whole(bn2), whole(att_W2), whole(att_b2),
            whole(read_W), whole(read_b), whole(h1_W), whole(h1_b),
            whole(h2_Ws), whole(h2_bs),
        ],
        out_specs=out_specs,
        out_shape=out_shape,
        scratch_shapes=[pltpu.VMEM((_L, 2 * _H), jnp.float32)],
        compiler_params=pltpu.CompilerParams(
            dimension_semantics=("arbitrary",)),
    )(x, x, Wn2, bn2, att_W2, att_b2, read_W, read_b, h1_W, h1_b,
      h2_Ws, h2_bs)


# speedup vs baseline: 1.2922x; 1.1656x over previous
import jax
import jax.numpy as jnp
from jax.experimental import pallas as pl
from jax.experimental.pallas import tpu as pltpu

_N, _F, _H, _L, _T = 100000, 128, 64, 4, 5
_BNR = 10000
_STEPS = (_N // 2) // _BNR


def _fused_kernel(xA_ref, xB_ref, Wn_ref, bn_ref, attW_ref, attb_ref,
                  readW_ref, readb_ref, h1W_ref, h1b_ref, h2W_ref, h2b_ref,
                  o0_ref, o1_ref, o2_ref, o3_ref, o4_ref, acc_ref):
    step = pl.program_id(0)

    @pl.when(step == 0)
    def _init():
        acc_ref[...] = jnp.zeros_like(acc_ref)

    x2 = jnp.concatenate([xA_ref[...], xB_ref[...]], axis=1)
    h = jnp.dot(x2, Wn_ref[...],
                preferred_element_type=jnp.float32) + bn_ref[...]
    for i in range(_L):
        h = jnp.dot(h, attW_ref[i], preferred_element_type=jnp.float32)
        h = jnp.maximum(h + attb_ref[i:i + 1, :], 0.0)
        acc_ref[i:i + 1, :] += jnp.sum(h, axis=0, keepdims=True)

    @pl.when(step == _STEPS - 1)
    def _readout():
        sums = acc_ref[:, :_H] + acc_ref[:, _H:]
        pooled = sums * (1.0 / _N)
        gr = jnp.sum(readb_ref[...], axis=0, keepdims=True)
        for i in range(_L):
            gr = gr + jnp.dot(pooled[i:i + 1, :], readW_ref[i],
                              preferred_element_type=jnp.float32)
        outs = (o0_ref, o1_ref, o2_ref, o3_ref, o4_ref)
        for j in range(_T):
            z = jnp.dot(gr, h1W_ref[j], preferred_element_type=jnp.float32)
            z = jnp.maximum(z + h1b_ref[j:j + 1, :], 0.0)
            o = (jnp.sum(z * h2W_ref[j:j + 1, :], axis=1, keepdims=True)
                 + h2b_ref[0:1, j:j + 1])
            if j in (0, 3, 4):
                o = jax.nn.sigmoid(o)
            outs[j][...] = o


def kernel(x, Wn, bn, att_W, att_b, read_W, read_b, h1_W, h1_b, h2_W, h2_b):
    eye2 = jnp.eye(2, dtype=x.dtype)
    Wn2 = jnp.kron(eye2, Wn)
    bn2 = jnp.tile(bn, 2).reshape(1, 2 * _H)
    att_W2 = jax.vmap(lambda w: jnp.kron(eye2, w))(att_W)
    att_b2 = jnp.tile(att_b, (1, 2))
    h2_Ws = h2_W[:, :, 0]
    h2_bs = h2_b.reshape(1, _T)

    whole = lambda a: pl.BlockSpec(a.shape, lambda i: (0,) * a.ndim)
    out_shape = tuple(jax.ShapeDtypeStruct((1, 1), jnp.float32)
                      for _ in range(_T))
    out_specs = tuple(pl.BlockSpec((1, 1), lambda i: (0, 0))
                      for _ in range(_T))

    return pl.pallas_call(
        _fused_kernel,
        grid=(_STEPS,),
        in_specs=[
            pl.BlockSpec((_BNR, _F), lambda i: (i, 0)),
            pl.BlockSpec((_BNR, _F), lambda i: (i + _STEPS, 0)),
            whole(Wn2), whole(bn2), whole(att_W2), whole(att_b2),
            whole(read_W), whole(read_b), whole(h1_W), whole(h1_b),
            whole(h2_Ws), whole(h2_bs),
        ],
        out_specs=out_specs,
        out_shape=out_shape,
        scratch_shapes=[pltpu.VMEM((_L, 2 * _H), jnp.float32)],
        compiler_params=pltpu.CompilerParams(
            dimension_semantics=("arbitrary",)),
    )(x, x, Wn2, bn2, att_W2, att_b2, read_W, read_b, h1_W, h1_b,
      h2_Ws, h2_bs)


# speedup vs baseline: 1.3026x; 1.0081x over previous
import jax
import jax.numpy as jnp
from jax.experimental import pallas as pl
from jax.experimental.pallas import tpu as pltpu

_N, _F, _H, _L, _T = 100000, 128, 64, 4, 5
_BNR = 10000
_STEPS = (_N // 2) // _BNR


def _fused_kernel(xA_ref, xB_ref, Wn_ref, bn_ref, attW_ref, attb_ref,
                  readW_ref, readb_ref, h1W_ref, h1b_ref, h2W_ref, h2b_ref,
                  o0_ref, o1_ref, o2_ref, o3_ref, o4_ref, acc_ref):
    step = pl.program_id(0)

    @pl.when(step == 0)
    def _init():
        acc_ref[...] = jnp.zeros_like(acc_ref)

    x2 = jnp.concatenate([xA_ref[...], xB_ref[...]], axis=1)
    h = jnp.dot(x2, Wn_ref[...], preferred_element_type=jnp.float32)
    h = jnp.maximum(h + bn_ref[...], 0.0)
    acc_ref[0:1, :] += jnp.sum(h, axis=0, keepdims=True)
    for i in range(1, _L):
        h = jnp.dot(h, attW_ref[i], preferred_element_type=jnp.float32)
        h = jnp.maximum(h + attb_ref[i:i + 1, :], 0.0)
        acc_ref[i:i + 1, :] += jnp.sum(h, axis=0, keepdims=True)

    @pl.when(step == _STEPS - 1)
    def _readout():
        sums = acc_ref[:, :_H] + acc_ref[:, _H:]
        pooled = sums * (1.0 / _N)
        gr = jnp.sum(readb_ref[...], axis=0, keepdims=True)
        for i in range(_L):
            gr = gr + jnp.dot(pooled[i:i + 1, :], readW_ref[i],
                              preferred_element_type=jnp.float32)
        outs = (o0_ref, o1_ref, o2_ref, o3_ref, o4_ref)
        for j in range(_T):
            z = jnp.dot(gr, h1W_ref[j], preferred_element_type=jnp.float32)
            z = jnp.maximum(z + h1b_ref[j:j + 1, :], 0.0)
            o = (jnp.sum(z * h2W_ref[j:j + 1, :], axis=1, keepdims=True)
                 + h2b_ref[0:1, j:j + 1])
            if j in (0, 3, 4):
                o = jax.nn.sigmoid(o)
            outs[j][...] = o


def kernel(x, Wn, bn, att_W, att_b, read_W, read_b, h1_W, h1_b, h2_W, h2_b):
    Wc = Wn @ att_W[0]
    bc = bn @ att_W[0] + att_b[0]
    eye2 = jnp.eye(2, dtype=x.dtype)
    Wn2 = jnp.kron(eye2, Wc)
    bn2 = jnp.tile(bc, 2).reshape(1, 2 * _H)
    att_W2 = jax.vmap(lambda w: jnp.kron(eye2, w))(att_W)
    att_b2 = jnp.tile(att_b, (1, 2))
    h2_Ws = h2_W[:, :, 0]
    h2_bs = h2_b.reshape(1, _T)

    whole = lambda a: pl.BlockSpec(a.shape, lambda i: (0,) * a.ndim)
    out_shape = tuple(jax.ShapeDtypeStruct((1, 1), jnp.float32)
                      for _ in range(_T))
    out_specs = tuple(pl.BlockSpec((1, 1), lambda i: (0, 0))
                      for _ in range(_T))

    return pl.pallas_call(
        _fused_kernel,
        grid=(_STEPS,),
        in_specs=[
            pl.BlockSpec((_BNR, _F), lambda i: (i, 0)),
            pl.BlockSpec((_BNR, _F), lambda i: (i + _STEPS, 0)),
            whole(Wn2), whole(bn2), whole(att_W2), whole(att_b2),
            whole(read_W), whole(read_b), whole(h1_W), whole(h1_b),
            whole(h2_Ws), whole(h2_bs),
        ],
        out_specs=out_specs,
        out_shape=out_shape,
        scratch_shapes=[pltpu.VMEM((_L, 2 * _H), jnp.float32)],
        compiler_params=pltpu.CompilerParams(
            dimension_semantics=("arbitrary",)),
    )(x, x, Wn2, bn2, att_W2, att_b2, read_W, read_b, h1_W, h1_b,
      h2_Ws, h2_bs)


# speedup vs baseline: 1.4194x; 1.0896x over previous
import jax
import jax.numpy as jnp
from jax.experimental import pallas as pl
from jax.experimental.pallas import tpu as pltpu

_N, _F, _H, _L, _T = 100000, 128, 64, 4, 5
_BNR = 10000
_STEPS = (_N // 2) // _BNR


def _blockdiag2(w):
    k = w.shape[0]
    z = jnp.zeros((k, k), jnp.float32)
    return jnp.concatenate(
        [jnp.concatenate([w, z], axis=1),
         jnp.concatenate([z, w], axis=1)], axis=0)


def _fused_kernel(xA_ref, xB_ref, Wn_ref, bn_ref, attW_ref, attb_ref,
                  readW_ref, readb_ref, h1W_ref, h1b_ref, h2W_ref, h2b_ref,
                  o0_ref, o1_ref, o2_ref, o3_ref, o4_ref,
                  acc_ref, Wc2_ref, bc2_ref, attW2_ref, attb2_ref):
    step = pl.program_id(0)

    @pl.when(step == 0)
    def _prep():
        acc_ref[...] = jnp.zeros_like(acc_ref)
        w0 = attW_ref[0]
        Wc = jnp.dot(Wn_ref[...], w0, preferred_element_type=jnp.float32)
        bc = (jnp.dot(bn_ref[...], w0, preferred_element_type=jnp.float32)
              + attb_ref[0:1, :])
        z = jnp.zeros((_F, _H), jnp.float32)
        Wc2_ref[...] = jnp.concatenate(
            [jnp.concatenate([Wc, z], axis=1),
             jnp.concatenate([z, Wc], axis=1)], axis=0)
        bc2_ref[...] = jnp.concatenate([bc, bc], axis=1)
        for i in range(1, _L):
            attW2_ref[i - 1] = _blockdiag2(attW_ref[i])
        attb2_ref[...] = jnp.concatenate(
            [attb_ref[...], attb_ref[...]], axis=1)

    x2 = jnp.concatenate([xA_ref[...], xB_ref[...]], axis=1)
    h = jnp.dot(x2, Wc2_ref[...], preferred_element_type=jnp.float32)
    h = jnp.maximum(h + bc2_ref[...], 0.0)
    acc_ref[0:1, :] += jnp.sum(h, axis=0, keepdims=True)
    for i in range(1, _L):
        h = jnp.dot(h, attW2_ref[i - 1], preferred_element_type=jnp.float32)
        h = jnp.maximum(h + attb2_ref[i:i + 1, :], 0.0)
        acc_ref[i:i + 1, :] += jnp.sum(h, axis=0, keepdims=True)

    @pl.when(step == _STEPS - 1)
    def _readout():
        sums = acc_ref[:, :_H] + acc_ref[:, _H:]
        pooled = sums * (1.0 / _N)
        gr = jnp.sum(readb_ref[...], axis=0, keepdims=True)
        for i in range(_L):
            gr = gr + jnp.dot(pooled[i:i + 1, :], readW_ref[i],
                              preferred_element_type=jnp.float32)
        outs = (o0_ref, o1_ref, o2_ref, o3_ref, o4_ref)
        for j in range(_T):
            z = jnp.dot(gr, h1W_ref[j], preferred_element_type=jnp.float32)
            z = jnp.maximum(z + h1b_ref[j:j + 1, :], 0.0)
            o = (jnp.dot(z, h2W_ref[j], preferred_element_type=jnp.float32)
                 + h2b_ref[j:j + 1, :])
            if j in (0, 3, 4):
                o = jax.nn.sigmoid(o)
            outs[j][...] = o


def kernel(x, Wn, bn, att_W, att_b, read_W, read_b, h1_W, h1_b, h2_W, h2_b):
    bn2 = bn.reshape(1, _H)

    whole = lambda a: pl.BlockSpec(a.shape, lambda i: (0,) * a.ndim)
    out_shape = tuple(jax.ShapeDtypeStruct((1, 1), jnp.float32)
                      for _ in range(_T))
    out_specs = tuple(pl.BlockSpec((1, 1), lambda i: (0, 0))
                      for _ in range(_T))

    return pl.pallas_call(
        _fused_kernel,
        grid=(_STEPS,),
        in_specs=[
            pl.BlockSpec((_BNR, _F), lambda i: (i, 0)),
            pl.BlockSpec((_BNR, _F), lambda i: (i + _STEPS, 0)),
            whole(Wn), whole(bn2), whole(att_W), whole(att_b),
            whole(read_W), whole(read_b), whole(h1_W), whole(h1_b),
            whole(h2_W), whole(h2_b),
        ],
        out_specs=out_specs,
        out_shape=out_shape,
        scratch_shapes=[
            pltpu.VMEM((_L, 2 * _H), jnp.float32),
            pltpu.VMEM((2 * _F, 2 * _H), jnp.float32),
            pltpu.VMEM((1, 2 * _H), jnp.float32),
            pltpu.VMEM((_L - 1, 2 * _H, 2 * _H), jnp.float32),
            pltpu.VMEM((_L, 2 * _H), jnp.float32),
        ],
        compiler_params=pltpu.CompilerParams(
            dimension_semantics=("arbitrary",)),
    )(x, x, Wn, bn2, att_W, att_b, read_W, read_b, h1_W, h1_b, h2_W, h2_b)
